# Initial kernel scaffold; baseline (speedup 1.0000x reference)
import jax
import jax.numpy as jnp
from jax.experimental import pallas as pl


def kernel(hidden_states, expert_affinities, expert_indices, seq_len, gate_up_proj, down_proj):
    raise NotImplementedError("write your pallas kernel here")


# speedup vs baseline: 1.6483x; 1.6483x over previous
import functools

import jax
import jax.numpy as jnp
from jax.experimental import pallas as pl
from jax.experimental.pallas import tpu as pltpu

E = 8
TOP_K = 2
H = 1024
I = 768
T_BLK = 1024


def _moe_dense_kernel(x_ref, aff_ref, idx_ref, gu_ref, dw_ref, out_ref):
    e = pl.program_id(1)

    x = x_ref[...]
    gu = jnp.dot(x, gu_ref[0], preferred_element_type=jnp.float32)
    gate = gu[:, :I]
    up = gu[:, I:]
    h = (jax.nn.sigmoid(gate) * gate * up).astype(jnp.bfloat16)
    y = jnp.dot(h, dw_ref[0], preferred_element_type=jnp.float32)

    idx = idx_ref[...]
    aff = aff_ref[...]
    lane = jax.lax.broadcasted_iota(jnp.int32, (1, E), 1)
    m0 = (idx[:, 0:1] == lane).astype(jnp.float32)
    m1 = (idx[:, 1:2] == lane).astype(jnp.float32)
    a0 = jnp.sum(m0 * aff, axis=1, keepdims=True)
    a1 = jnp.sum(m1 * aff, axis=1, keepdims=True)
    denom = a0 + a1 + 1e-9
    w = ((idx[:, 0:1] == e) * a0 + (idx[:, 1:2] == e) * a1) / denom

    contrib = y * w
    @pl.when(e == 0)
    def _():
        out_ref[...] = contrib

    @pl.when(e != 0)
    def _():
        out_ref[...] += contrib


@jax.jit
def kernel(hidden_states, expert_affinities, expert_indices, seq_len,
           gate_up_proj, down_proj):
    del seq_len
    T = hidden_states.shape[0]
    nt = T // T_BLK

    x_bf = hidden_states.astype(jnp.bfloat16)
    gu_bf = gate_up_proj.astype(jnp.bfloat16)
    dw_bf = down_proj.astype(jnp.bfloat16)

    out = pl.pallas_call(
        _moe_dense_kernel,
        grid=(nt, E),
        in_specs=[
            pl.BlockSpec((T_BLK, H), lambda t, e: (t, 0)),
            pl.BlockSpec((T_BLK, E), lambda t, e: (t, 0)),
            pl.BlockSpec((T_BLK, TOP_K), lambda t, e: (t, 0)),
            pl.BlockSpec((1, H, 2 * I), lambda t, e: (e, 0, 0)),
            pl.BlockSpec((1, I, H), lambda t, e: (e, 0, 0)),
        ],
        out_specs=pl.BlockSpec((T_BLK, H), lambda t, e: (t, 0)),
        out_shape=jax.ShapeDtypeStruct((T, H), jnp.float32),
        compiler_params=pltpu.CompilerParams(
            dimension_semantics=("parallel", "arbitrary"),
        ),
    )(x_bf, expert_affinities, expert_indices, gu_bf, dw_bf)
    return out
